# baseline (speedup 1.0000x reference)
import functools

import jax
import jax.numpy as jnp
from jax import lax
from jax.experimental import pallas as pl
from jax.experimental.pallas import tpu as pltpu
from jax.experimental.pallas import tpu_sc as plsc

D = 128
N_NODES = 10000
N_EDGES = 320000
N_REL = 38

NC = 2
NS = 16
NW = NC * NS
L = 16

E_PER_W = N_EDGES // NW
E_CHUNK = 80
N_CHUNKS = E_PER_W // E_CHUNK
NBUF = 5
N_GROUPS = N_CHUNKS // NBUF

M_CHUNK = 40
M_N_CHUNKS = E_PER_W // M_CHUNK
M_GROUPS = M_N_CHUNKS // NBUF

AGG_ROWS = 10240
ROWS_PER_TILE = AGG_ROWS // NS

GB = 10240
GB_PER_W = GB // NW
G_CHUNK = 80

_mesh = plsc.VectorSubcoreMesh(core_axis_name="c", subcore_axis_name="s")


def _wid():
    return lax.axis_index("s") * NC + lax.axis_index("c")


@functools.partial(
    pl.kernel,
    out_type=jax.ShapeDtypeStruct((GB, D), jnp.float32),
    mesh=_mesh,
    scratch_types=[
        pltpu.VMEM((GB_PER_W,), jnp.int32),
        pltpu.VMEM((G_CHUNK, D), jnp.float32),
        pltpu.SemaphoreType.DMA,
    ],
)
def _sc_gather_rows(table_hbm, idx_hbm, out_hbm, idx_v, rows_v, sem):
    wid = _wid()
    base = wid * GB_PER_W
    pltpu.sync_copy(idx_hbm.at[pl.ds(base, GB_PER_W)], idx_v)

    def body(c, _):
        pltpu.async_copy(
            table_hbm.at[idx_v.at[pl.ds(c * G_CHUNK, G_CHUNK)]], rows_v, sem
        ).wait()
        pltpu.sync_copy(rows_v, out_hbm.at[pl.ds(base + c * G_CHUNK, G_CHUNK)])
        return 0

    lax.fori_loop(0, GB_PER_W // G_CHUNK, body, 0)


def _tc_pre_body(x_ref, wm1_ref, wm2_ref, l2_ref, rel_ref, bm_ref, xm_ref,
                 rw2_ref, rl2_ref):
    xm_ref[...] = jnp.dot(x_ref[...], wm1_ref[...],
                          preferred_element_type=jnp.float32) + bm_ref[...]
    rw2_ref[...] = jnp.dot(rel_ref[...], wm2_ref[...],
                           preferred_element_type=jnp.float32)
    rl2_ref[...] = jnp.dot(rel_ref[...], l2_ref[...],
                           preferred_element_type=jnp.float32)


def _tc_pre(x, wm1, wm2, l2, rel_emb, bm):
    return pl.pallas_call(
        _tc_pre_body,
        grid=(10,),
        in_specs=[
            pl.BlockSpec((1000, D), lambda i: (i, 0)),
            pl.BlockSpec((D, D), lambda i: (0, 0)),
            pl.BlockSpec((D, D), lambda i: (0, 0)),
            pl.BlockSpec((D, D), lambda i: (0, 0)),
            pl.BlockSpec((N_REL, D), lambda i: (0, 0)),
            pl.BlockSpec((1, D), lambda i: (0, 0)),
        ],
        out_specs=[
            pl.BlockSpec((1000, D), lambda i: (i, 0)),
            pl.BlockSpec((N_REL, D), lambda i: (0, 0)),
            pl.BlockSpec((N_REL, D), lambda i: (0, 0)),
        ],
        out_shape=[
            jax.ShapeDtypeStruct((N_NODES, D), jnp.float32),
            jax.ShapeDtypeStruct((N_REL, D), jnp.float32),
            jax.ShapeDtypeStruct((N_REL, D), jnp.float32),
        ],
    )(x, wm1, wm2, l2, rel_emb, bm)


@functools.partial(
    pl.kernel,
    out_type=jax.ShapeDtypeStruct((NC, AGG_ROWS, D), jnp.float32),
    mesh=_mesh,
    scratch_types=[
        pltpu.VMEM((E_PER_W,), jnp.int32),
        [pltpu.VMEM((M_CHUNK,), jnp.int32) for _ in range(NBUF)],
        [pltpu.VMEM((M_CHUNK,), jnp.int32) for _ in range(NBUF)],
        [pltpu.VMEM((M_CHUNK,), jnp.float32) for _ in range(NBUF)],
        [pltpu.VMEM((M_CHUNK, D), jnp.float32) for _ in range(NBUF)],
        pltpu.VMEM((M_CHUNK, D), jnp.float32),
        pltpu.VMEM((N_REL * D,), jnp.float32),
        pltpu.VMEM_SHARED((AGG_ROWS, D), jnp.float32),
        [pltpu.SemaphoreType.DMA for _ in range(NBUF)],
        [pltpu.SemaphoreType.DMA for _ in range(NBUF)],
    ],
    compiler_params=pltpu.CompilerParams(needs_layout_passes=False),
)
def _sc_message(src_hbm, dst_hbm, rel_hbm, w_hbm, xm_hbm, rw2_hbm,
                zeros_hbm, out_hbm, src_v, dst_v, rel_v, w_v, rows_v, msg_v,
                rw2_v, agg_sh, gsem, ssem):
    core = lax.axis_index("c")
    sid = lax.axis_index("s")
    wid = sid * NC + core
    ebase = wid * E_PER_W

    pltpu.sync_copy(rw2_hbm, rw2_v)
    pltpu.sync_copy(src_hbm.at[pl.ds(ebase, E_PER_W)], src_v)
    pltpu.sync_copy(zeros_hbm, agg_sh.at[pl.ds(sid * ROWS_PER_TILE,
                                               ROWS_PER_TILE)])

    def issue(c, b):
        base = ebase + c * M_CHUNK
        pltpu.async_copy(xm_hbm.at[src_v.at[pl.ds(c * M_CHUNK, M_CHUNK)]],
                         rows_v[b], gsem[b])
        pltpu.async_copy(dst_hbm.at[pl.ds(base, M_CHUNK)], dst_v[b], ssem[b])
        pltpu.async_copy(rel_hbm.at[pl.ds(base, M_CHUNK)], rel_v[b], ssem[b])
        pltpu.async_copy(w_hbm.at[pl.ds(base, M_CHUNK)], w_v[b], ssem[b])

    for b in range(NBUF):
        issue(b, b)
    plsc.subcore_barrier()

    iota = lax.broadcasted_iota(jnp.int32, (L,), 0)

    def group_body(g, _):
        for b in range(NBUF):
            c = g * NBUF + b
            pltpu.make_async_copy(
                xm_hbm.at[src_v.at[pl.ds(c * M_CHUNK, M_CHUNK)]], rows_v[b],
                gsem[b]).wait()
            pltpu.make_async_copy(dst_hbm.at[pl.ds(0, M_CHUNK)], dst_v[b],
                                  ssem[b]).wait()
            pltpu.make_async_copy(rel_hbm.at[pl.ds(0, M_CHUNK)], rel_v[b],
                                  ssem[b]).wait()
            pltpu.make_async_copy(w_hbm.at[pl.ds(0, M_CHUNK)], w_v[b],
                                  ssem[b]).wait()

            @plsc.parallel_loop(0, M_CHUNK, unroll=8)
            def edge_body(e):
                ev = jnp.broadcast_to(e, (L,)).astype(jnp.int32)
                wv = plsc.load_gather(w_v[b], [ev])
                rv = plsc.load_gather(rel_v[b], [ev])
                rbase = rv * D + iota
                for j in range(D // L):
                    xmv = rows_v[b][e, pl.ds(j * L, L)]
                    relv = plsc.load_gather(rw2_v, [rbase + (j * L)])
                    msg_v[e, pl.ds(j * L, L)] = jnp.maximum(
                        xmv + wv * relv, 0.0)

            pltpu.sync_copy(msg_v, agg_sh.at[dst_v[b]], add=True)

            c5 = c + NBUF

            @pl.when(c5 < M_N_CHUNKS)
            def _():
                issue(c5, b)

        return 0

    lax.fori_loop(0, M_GROUPS, group_body, 0)
    plsc.subcore_barrier()
    pltpu.sync_copy(
        agg_sh.at[pl.ds(sid * ROWS_PER_TILE, ROWS_PER_TILE)],
        out_hbm.at[core].at[pl.ds(sid * ROWS_PER_TILE, ROWS_PER_TILE)],
    )


def _tc_post_body(parts_ref, x_ref, wu_ref, ws_ref, bu_ref, l1_ref, l3_ref,
                  bl_ref, xs1_ref, xs3_ref):
    agg = parts_ref[0] + parts_ref[1]
    x_out = jax.nn.relu(
        jnp.dot(agg, wu_ref[...], preferred_element_type=jnp.float32)
        + jnp.dot(x_ref[...], ws_ref[...], preferred_element_type=jnp.float32)
        + bu_ref[...]
    )
    xs1_ref[...] = jnp.dot(x_out, l1_ref[...],
                           preferred_element_type=jnp.float32) + bl_ref[...]
    xs3_ref[...] = jnp.dot(x_out, l3_ref[...],
                           preferred_element_type=jnp.float32)


def _tc_post(parts, x, wu, ws, bu, l1, l3, bl):
    return pl.pallas_call(
        _tc_post_body,
        grid=(10,),
        in_specs=[
            pl.BlockSpec((NC, 1000, D), lambda i: (0, i, 0)),
            pl.BlockSpec((1000, D), lambda i: (i, 0)),
            pl.BlockSpec((D, D), lambda i: (0, 0)),
            pl.BlockSpec((D, D), lambda i: (0, 0)),
            pl.BlockSpec((1, D), lambda i: (0, 0)),
            pl.BlockSpec((D, D), lambda i: (0, 0)),
            pl.BlockSpec((D, D), lambda i: (0, 0)),
            pl.BlockSpec((1, D), lambda i: (0, 0)),
        ],
        out_specs=[
            pl.BlockSpec((1000, D), lambda i: (i, 0)),
            pl.BlockSpec((1000, D), lambda i: (i, 0)),
        ],
        out_shape=[
            jax.ShapeDtypeStruct((N_NODES, D), jnp.float32),
            jax.ShapeDtypeStruct((N_NODES, D), jnp.float32),
        ],
    )(parts, x, wu, ws, bu, l1, l3, bl)


@functools.partial(
    pl.kernel,
    out_type=[
        jax.ShapeDtypeStruct((N_EDGES, D), jnp.float32),
        jax.ShapeDtypeStruct((N_EDGES,), jnp.int32),
        jax.ShapeDtypeStruct((N_EDGES,), jnp.int32),
    ],
    mesh=_mesh,
    scratch_types=[
        [pltpu.VMEM((E_CHUNK,), jnp.int32) for _ in range(NBUF)],
        [pltpu.VMEM((E_CHUNK,), jnp.int32) for _ in range(NBUF)],
        [pltpu.VMEM((E_CHUNK,), jnp.int32) for _ in range(NBUF)],
        [pltpu.VMEM((E_CHUNK,), jnp.float32) for _ in range(NBUF)],
        [pltpu.VMEM((E_CHUNK, D), jnp.float32) for _ in range(NBUF)],
        [pltpu.VMEM((E_CHUNK, D), jnp.float32) for _ in range(NBUF)],
        pltpu.VMEM((N_REL * D,), jnp.float32),
        pltpu.VMEM((N_NODES,), jnp.int32),
        [pltpu.VMEM((E_CHUNK,), jnp.int32) for _ in range(NBUF)],
        [pltpu.VMEM((E_CHUNK,), jnp.int32) for _ in range(NBUF)],
        [pltpu.SemaphoreType.DMA for _ in range(NBUF)],
        [pltpu.SemaphoreType.DMA for _ in range(NBUF)],
        [pltpu.SemaphoreType.DMA for _ in range(NBUF)],
        [pltpu.SemaphoreType.DMA for _ in range(NBUF)],
    ],
    compiler_params=pltpu.CompilerParams(needs_layout_passes=False),
)
def _sc_encode(src_hbm, dst_hbm, rel_hbm, w_hbm, xs1_hbm, xs3_hbm, rl2_hbm,
               cid_hbm, out_hbm, osrc_hbm, odst_hbm, src_v, dst_v, rel_v,
               w_v, r1_v, r3_v, rl2_v, cid_v, cs_v, cd_v, g1sem, g3sem,
               ssem, osem):
    wid = _wid()
    ebase = wid * E_PER_W

    pltpu.sync_copy(rl2_hbm, rl2_v)
    pltpu.sync_copy(cid_hbm, cid_v)

    def issue(c, b):
        base = ebase + c * E_CHUNK
        pltpu.sync_copy(src_hbm.at[pl.ds(base, E_CHUNK)], src_v[b])
        pltpu.sync_copy(dst_hbm.at[pl.ds(base, E_CHUNK)], dst_v[b])
        pltpu.async_copy(xs1_hbm.at[src_v[b]], r1_v[b], g1sem[b])
        pltpu.async_copy(xs3_hbm.at[dst_v[b]], r3_v[b], g3sem[b])
        pltpu.async_copy(rel_hbm.at[pl.ds(base, E_CHUNK)], rel_v[b], ssem[b])
        pltpu.async_copy(w_hbm.at[pl.ds(base, E_CHUNK)], w_v[b], ssem[b])

    def wait_writes(c, b):
        base = ebase + c * E_CHUNK
        pltpu.make_async_copy(r3_v[b], out_hbm.at[pl.ds(base, E_CHUNK)],
                              osem[b]).wait()
        pltpu.make_async_copy(cs_v[b], osrc_hbm.at[pl.ds(base, E_CHUNK)],
                              osem[b]).wait()
        pltpu.make_async_copy(cd_v[b], odst_hbm.at[pl.ds(base, E_CHUNK)],
                              osem[b]).wait()

    for b in range(NBUF - 1):
        issue(b, b)

    iota = lax.broadcasted_iota(jnp.int32, (L,), 0)

    def group_body(g, _):
        for b in range(NBUF):
            c = g * NBUF + b
            ca = c + NBUF - 1
            ba = (b + NBUF - 1) % NBUF

            @pl.when(ca < N_CHUNKS)
            def _():
                @pl.when(c >= 1)
                def _():
                    wait_writes(c - 1, ba)

                issue(ca, ba)

            pltpu.make_async_copy(xs1_hbm.at[src_v[b]], r1_v[b],
                                  g1sem[b]).wait()
            pltpu.make_async_copy(xs3_hbm.at[dst_v[b]], r3_v[b],
                                  g3sem[b]).wait()
            pltpu.make_async_copy(rel_hbm.at[pl.ds(0, E_CHUNK)], rel_v[b],
                                  ssem[b]).wait()
            pltpu.make_async_copy(w_hbm.at[pl.ds(0, E_CHUNK)], w_v[b],
                                  ssem[b]).wait()

            @plsc.parallel_loop(0, E_CHUNK, unroll=8)
            def edge_body(e):
                ev = jnp.broadcast_to(e, (L,)).astype(jnp.int32)
                wv = plsc.load_gather(w_v[b], [ev])
                rv = plsc.load_gather(rel_v[b], [ev])
                rbase = rv * D + iota
                for j in range(D // L):
                    a = r1_v[b][e, pl.ds(j * L, L)]
                    bb = r3_v[b][e, pl.ds(j * L, L)]
                    relv = plsc.load_gather(rl2_v, [rbase + (j * L)])
                    r3_v[b][e, pl.ds(j * L, L)] = a + wv * relv + bb

            for k in range(E_CHUNK // L):
                sv = src_v[b][pl.ds(k * L, L)]
                dv = dst_v[b][pl.ds(k * L, L)]
                cs_v[b][pl.ds(k * L, L)] = plsc.load_gather(cid_v, [sv])
                cd_v[b][pl.ds(k * L, L)] = plsc.load_gather(cid_v, [dv])

            base_c = ebase + c * E_CHUNK
            pltpu.async_copy(r3_v[b], out_hbm.at[pl.ds(base_c, E_CHUNK)],
                             osem[b])
            pltpu.async_copy(cs_v[b], osrc_hbm.at[pl.ds(base_c, E_CHUNK)],
                             osem[b])
            pltpu.async_copy(cd_v[b], odst_hbm.at[pl.ds(base_c, E_CHUNK)],
                             osem[b])

        return 0

    lax.fori_loop(0, N_GROUPS, group_body, 0)
    for b in range(NBUF):
        wait_writes(N_CHUNKS - NBUF + b, b)


@jax.jit
def _run(concept_ids, edge_index, edge_attr, concept_embedding,
         relation_embedding, W_msg, b_msg, W_self, W_upd, b_upd, W_lin,
         b_lin):
    src = edge_index[0]
    dst = edge_index[1]
    rel = edge_attr[:, 0].astype(jnp.int32)
    w = edge_attr[:, 1]

    idx_pad = jnp.concatenate(
        [concept_ids, jnp.zeros((GB - N_NODES,), jnp.int32)])
    x = _sc_gather_rows(concept_embedding, idx_pad)[:N_NODES]

    xm, rw2, rl2 = _tc_pre(x, W_msg[:D], W_msg[D:], W_lin[D:2 * D],
                           relation_embedding, b_msg.reshape(1, D))

    zeros = jnp.zeros((ROWS_PER_TILE, D), jnp.float32)
    parts = _sc_message(src, dst, rel, w, xm, rw2.reshape(-1), zeros)

    xs1, xs3 = _tc_post(parts, x, W_upd, W_self, b_upd.reshape(1, D),
                        W_lin[:D], W_lin[2 * D:], b_lin.reshape(1, D))

    enc, cs, cd = _sc_encode(src, dst, rel, w, xs1, xs3, rl2.reshape(-1),
                             concept_ids)
    triple_ids = jnp.stack([cs, rel, cd], axis=1)
    return enc, triple_ids


def kernel(concept_ids, edge_index, edge_attr, concept_embedding,
           relation_embedding, W_msg, b_msg, W_self, W_upd, b_upd, W_lin,
           b_lin):
    return _run(concept_ids, edge_index, edge_attr, concept_embedding,
                relation_embedding, W_msg, b_msg, W_self, W_upd, b_upd,
                W_lin, b_lin)


# speedup vs baseline: 1.2304x; 1.2304x over previous
import functools

import jax
import jax.numpy as jnp
from jax import lax
from jax.experimental import pallas as pl
from jax.experimental.pallas import tpu as pltpu
from jax.experimental.pallas import tpu_sc as plsc

D = 128
N_NODES = 10000
N_EDGES = 320000
N_REL = 38

NC = 2
NS = 16
NW = NC * NS
L = 16

E_PER_W = N_EDGES // NW
E_CHUNK = 80
N_CHUNKS = E_PER_W // E_CHUNK
NBUF = 5
N_GROUPS = N_CHUNKS // NBUF

M_CHUNK = 40
M_N_CHUNKS = E_PER_W // M_CHUNK
M_GROUPS = M_N_CHUNKS // NBUF

AGG_ROWS = 10240
ROWS_PER_TILE = AGG_ROWS // NS

GB = 10240
GB_PER_W = GB // NW
G_CHUNK = 80

_mesh = plsc.VectorSubcoreMesh(core_axis_name="c", subcore_axis_name="s")


def _wid():
    return lax.axis_index("s") * NC + lax.axis_index("c")


@functools.partial(
    pl.kernel,
    out_type=jax.ShapeDtypeStruct((GB, D), jnp.float32),
    mesh=_mesh,
    scratch_types=[
        pltpu.VMEM((GB_PER_W,), jnp.int32),
        pltpu.VMEM((G_CHUNK, D), jnp.float32),
        pltpu.SemaphoreType.DMA,
    ],
)
def _sc_gather_rows(table_hbm, idx_hbm, out_hbm, idx_v, rows_v, sem):
    wid = _wid()
    base = wid * GB_PER_W
    pltpu.sync_copy(idx_hbm.at[pl.ds(base, GB_PER_W)], idx_v)

    def body(c, _):
        pltpu.async_copy(
            table_hbm.at[idx_v.at[pl.ds(c * G_CHUNK, G_CHUNK)]], rows_v, sem
        ).wait()
        pltpu.sync_copy(rows_v, out_hbm.at[pl.ds(base + c * G_CHUNK, G_CHUNK)])
        return 0

    lax.fori_loop(0, GB_PER_W // G_CHUNK, body, 0)


def _tc_pre_body(x_ref, wm1_ref, wm2_ref, l2_ref, rel_ref, bm_ref, xm_ref,
                 rw2_ref, rl2_ref):
    xm_ref[...] = jnp.dot(x_ref[...], wm1_ref[...],
                          preferred_element_type=jnp.float32) + bm_ref[...]
    rw2_ref[...] = jnp.dot(rel_ref[...], wm2_ref[...],
                           preferred_element_type=jnp.float32)
    rl2_ref[...] = jnp.dot(rel_ref[...], l2_ref[...],
                           preferred_element_type=jnp.float32)


def _tc_pre(x, wm1, wm2, l2, rel_emb, bm):
    return pl.pallas_call(
        _tc_pre_body,
        grid=(10,),
        in_specs=[
            pl.BlockSpec((1000, D), lambda i: (i, 0)),
            pl.BlockSpec((D, D), lambda i: (0, 0)),
            pl.BlockSpec((D, D), lambda i: (0, 0)),
            pl.BlockSpec((D, D), lambda i: (0, 0)),
            pl.BlockSpec((N_REL, D), lambda i: (0, 0)),
            pl.BlockSpec((1, D), lambda i: (0, 0)),
        ],
        out_specs=[
            pl.BlockSpec((1000, D), lambda i: (i, 0)),
            pl.BlockSpec((N_REL, D), lambda i: (0, 0)),
            pl.BlockSpec((N_REL, D), lambda i: (0, 0)),
        ],
        out_shape=[
            jax.ShapeDtypeStruct((N_NODES, D), jnp.float32),
            jax.ShapeDtypeStruct((N_REL, D), jnp.float32),
            jax.ShapeDtypeStruct((N_REL, D), jnp.float32),
        ],
    )(x, wm1, wm2, l2, rel_emb, bm)


@functools.partial(
    pl.kernel,
    out_type=jax.ShapeDtypeStruct((NC, AGG_ROWS, D), jnp.float32),
    mesh=_mesh,
    scratch_types=[
        pltpu.VMEM((E_PER_W,), jnp.int32),
        [pltpu.VMEM((M_CHUNK,), jnp.int32) for _ in range(NBUF)],
        [pltpu.VMEM((M_CHUNK,), jnp.int32) for _ in range(NBUF)],
        [pltpu.VMEM((M_CHUNK,), jnp.float32) for _ in range(NBUF)],
        [pltpu.VMEM((M_CHUNK, D), jnp.float32) for _ in range(NBUF)],
        pltpu.VMEM((M_CHUNK, D), jnp.float32),
        pltpu.VMEM((N_REL * D,), jnp.float32),
        pltpu.VMEM_SHARED((AGG_ROWS, D), jnp.float32),
        [pltpu.SemaphoreType.DMA for _ in range(NBUF)],
        [pltpu.SemaphoreType.DMA for _ in range(NBUF)],
    ],
    compiler_params=pltpu.CompilerParams(needs_layout_passes=False),
)
def _sc_message(src_hbm, dst_hbm, rel_hbm, w_hbm, xm_hbm, rw2_hbm,
                zeros_hbm, out_hbm, src_v, dst_v, rel_v, w_v, rows_v, msg_v,
                rw2_v, agg_sh, gsem, ssem):
    core = lax.axis_index("c")
    sid = lax.axis_index("s")
    wid = sid * NC + core
    ebase = wid * E_PER_W

    pltpu.sync_copy(rw2_hbm, rw2_v)
    pltpu.sync_copy(src_hbm.at[pl.ds(ebase, E_PER_W)], src_v)
    pltpu.sync_copy(zeros_hbm, agg_sh.at[pl.ds(sid * ROWS_PER_TILE,
                                               ROWS_PER_TILE)])

    def issue(c, b):
        base = ebase + c * M_CHUNK
        pltpu.async_copy(xm_hbm.at[src_v.at[pl.ds(c * M_CHUNK, M_CHUNK)]],
                         rows_v[b], gsem[b])
        pltpu.async_copy(dst_hbm.at[pl.ds(base, M_CHUNK)], dst_v[b], ssem[b])
        pltpu.async_copy(rel_hbm.at[pl.ds(base, M_CHUNK)], rel_v[b], ssem[b])
        pltpu.async_copy(w_hbm.at[pl.ds(base, M_CHUNK)], w_v[b], ssem[b])

    for b in range(NBUF):
        issue(b, b)
    plsc.subcore_barrier()

    iota = lax.broadcasted_iota(jnp.int32, (L,), 0)

    def group_body(g, _):
        for b in range(NBUF):
            c = g * NBUF + b
            pltpu.make_async_copy(
                xm_hbm.at[src_v.at[pl.ds(c * M_CHUNK, M_CHUNK)]], rows_v[b],
                gsem[b]).wait()
            pltpu.make_async_copy(dst_hbm.at[pl.ds(0, M_CHUNK)], dst_v[b],
                                  ssem[b]).wait()
            pltpu.make_async_copy(rel_hbm.at[pl.ds(0, M_CHUNK)], rel_v[b],
                                  ssem[b]).wait()
            pltpu.make_async_copy(w_hbm.at[pl.ds(0, M_CHUNK)], w_v[b],
                                  ssem[b]).wait()

            @plsc.parallel_loop(0, M_CHUNK, unroll=4)
            def edge_body(e):
                ev = jnp.broadcast_to(e, (L,)).astype(jnp.int32)
                wv = plsc.load_gather(w_v[b], [ev])
                rv = plsc.load_gather(rel_v[b], [ev])
                rbase = rv * D + iota
                for j in range(D // L):
                    xmv = rows_v[b][e, pl.ds(j * L, L)]
                    relv = plsc.load_gather(rw2_v, [rbase + (j * L)])
                    msg_v[e, pl.ds(j * L, L)] = jnp.maximum(
                        xmv + wv * relv, 0.0)

            pltpu.sync_copy(msg_v, agg_sh.at[dst_v[b]], add=True)

            c5 = c + NBUF

            @pl.when(c5 < M_N_CHUNKS)
            def _():
                issue(c5, b)

        return 0

    lax.fori_loop(0, M_GROUPS, group_body, 0)
    plsc.subcore_barrier()
    pltpu.sync_copy(
        agg_sh.at[pl.ds(sid * ROWS_PER_TILE, ROWS_PER_TILE)],
        out_hbm.at[core].at[pl.ds(sid * ROWS_PER_TILE, ROWS_PER_TILE)],
    )


def _tc_post_body(parts_ref, x_ref, wu_ref, ws_ref, bu_ref, l1_ref, l3_ref,
                  bl_ref, xs1_ref, xs3_ref):
    agg = parts_ref[0] + parts_ref[1]
    x_out = jax.nn.relu(
        jnp.dot(agg, wu_ref[...], preferred_element_type=jnp.float32)
        + jnp.dot(x_ref[...], ws_ref[...], preferred_element_type=jnp.float32)
        + bu_ref[...]
    )
    xs1_ref[...] = jnp.dot(x_out, l1_ref[...],
                           preferred_element_type=jnp.float32) + bl_ref[...]
    xs3_ref[...] = jnp.dot(x_out, l3_ref[...],
                           preferred_element_type=jnp.float32)


def _tc_post(parts, x, wu, ws, bu, l1, l3, bl):
    return pl.pallas_call(
        _tc_post_body,
        grid=(10,),
        in_specs=[
            pl.BlockSpec((NC, 1000, D), lambda i: (0, i, 0)),
            pl.BlockSpec((1000, D), lambda i: (i, 0)),
            pl.BlockSpec((D, D), lambda i: (0, 0)),
            pl.BlockSpec((D, D), lambda i: (0, 0)),
            pl.BlockSpec((1, D), lambda i: (0, 0)),
            pl.BlockSpec((D, D), lambda i: (0, 0)),
            pl.BlockSpec((D, D), lambda i: (0, 0)),
            pl.BlockSpec((1, D), lambda i: (0, 0)),
        ],
        out_specs=[
            pl.BlockSpec((1000, D), lambda i: (i, 0)),
            pl.BlockSpec((1000, D), lambda i: (i, 0)),
        ],
        out_shape=[
            jax.ShapeDtypeStruct((N_NODES, D), jnp.float32),
            jax.ShapeDtypeStruct((N_NODES, D), jnp.float32),
        ],
    )(parts, x, wu, ws, bu, l1, l3, bl)


@functools.partial(
    pl.kernel,
    out_type=[
        jax.ShapeDtypeStruct((N_EDGES, D), jnp.float32),
        jax.ShapeDtypeStruct((N_EDGES,), jnp.int32),
        jax.ShapeDtypeStruct((N_EDGES,), jnp.int32),
    ],
    mesh=_mesh,
    scratch_types=[
        pltpu.VMEM((E_PER_W,), jnp.int32),
        [pltpu.VMEM((E_CHUNK,), jnp.int32) for _ in range(NBUF)],
        [pltpu.VMEM((E_CHUNK,), jnp.int32) for _ in range(NBUF)],
        [pltpu.VMEM((E_CHUNK,), jnp.float32) for _ in range(NBUF)],
        [pltpu.VMEM((E_CHUNK, D), jnp.float32) for _ in range(NBUF)],
        [pltpu.VMEM((E_CHUNK, D), jnp.float32) for _ in range(NBUF)],
        pltpu.VMEM((N_REL * D,), jnp.float32),
        pltpu.VMEM((N_NODES,), jnp.int32),
        [pltpu.VMEM((E_CHUNK,), jnp.int32) for _ in range(NBUF)],
        [pltpu.VMEM((E_CHUNK,), jnp.int32) for _ in range(NBUF)],
        [pltpu.SemaphoreType.DMA for _ in range(NBUF)],
        [pltpu.SemaphoreType.DMA for _ in range(NBUF)],
        [pltpu.SemaphoreType.DMA for _ in range(NBUF)],
        [pltpu.SemaphoreType.DMA for _ in range(NBUF)],
    ],
    compiler_params=pltpu.CompilerParams(needs_layout_passes=False),
)
def _sc_encode(src_hbm, dst_hbm, rel_hbm, w_hbm, xs1_hbm, xs3_hbm, rl2_hbm,
               cid_hbm, out_hbm, osrc_hbm, odst_hbm, src_v, dst_v, rel_v,
               w_v, r1_v, r3_v, rl2_v, cid_v, cs_v, cd_v, g1sem, g3sem,
               ssem, osem):
    wid = _wid()
    ebase = wid * E_PER_W

    pltpu.sync_copy(rl2_hbm, rl2_v)
    pltpu.sync_copy(cid_hbm, cid_v)
    pltpu.sync_copy(src_hbm.at[pl.ds(ebase, E_PER_W)], src_v)

    def issue(c, b):
        base = ebase + c * E_CHUNK
        pltpu.sync_copy(dst_hbm.at[pl.ds(base, E_CHUNK)], dst_v[b])
        pltpu.async_copy(xs1_hbm.at[src_v.at[pl.ds(c * E_CHUNK, E_CHUNK)]],
                         r1_v[b], g1sem[b])
        pltpu.async_copy(xs3_hbm.at[dst_v[b]], r3_v[b], g3sem[b])
        pltpu.async_copy(rel_hbm.at[pl.ds(base, E_CHUNK)], rel_v[b], ssem[b])
        pltpu.async_copy(w_hbm.at[pl.ds(base, E_CHUNK)], w_v[b], ssem[b])

    def wait_writes(c, b):
        base = ebase + c * E_CHUNK
        pltpu.make_async_copy(r3_v[b], out_hbm.at[pl.ds(base, E_CHUNK)],
                              osem[b]).wait()
        pltpu.make_async_copy(cs_v[b], osrc_hbm.at[pl.ds(base, E_CHUNK)],
                              osem[b]).wait()
        pltpu.make_async_copy(cd_v[b], odst_hbm.at[pl.ds(base, E_CHUNK)],
                              osem[b]).wait()

    for b in range(NBUF - 1):
        issue(b, b)

    iota = lax.broadcasted_iota(jnp.int32, (L,), 0)

    def group_body(g, _):
        for b in range(NBUF):
            c = g * NBUF + b
            ca = c + NBUF - 1
            ba = (b + NBUF - 1) % NBUF

            @pl.when(ca < N_CHUNKS)
            def _():
                @pl.when(c >= 1)
                def _():
                    wait_writes(c - 1, ba)

                issue(ca, ba)

            pltpu.make_async_copy(
                xs1_hbm.at[src_v.at[pl.ds(c * E_CHUNK, E_CHUNK)]], r1_v[b],
                g1sem[b]).wait()
            pltpu.make_async_copy(xs3_hbm.at[dst_v[b]], r3_v[b],
                                  g3sem[b]).wait()
            pltpu.make_async_copy(rel_hbm.at[pl.ds(0, E_CHUNK)], rel_v[b],
                                  ssem[b]).wait()
            pltpu.make_async_copy(w_hbm.at[pl.ds(0, E_CHUNK)], w_v[b],
                                  ssem[b]).wait()

            @plsc.parallel_loop(0, E_CHUNK, unroll=4)
            def edge_body(e):
                ev = jnp.broadcast_to(e, (L,)).astype(jnp.int32)
                wv = plsc.load_gather(w_v[b], [ev])
                rv = plsc.load_gather(rel_v[b], [ev])
                rbase = rv * D + iota
                for j in range(D // L):
                    a = r1_v[b][e, pl.ds(j * L, L)]
                    bb = r3_v[b][e, pl.ds(j * L, L)]
                    relv = plsc.load_gather(rl2_v, [rbase + (j * L)])
                    r3_v[b][e, pl.ds(j * L, L)] = a + wv * relv + bb

            for k in range(E_CHUNK // L):
                sv = src_v[pl.ds(c * E_CHUNK + k * L, L)]
                dv = dst_v[b][pl.ds(k * L, L)]
                cs_v[b][pl.ds(k * L, L)] = plsc.load_gather(cid_v, [sv])
                cd_v[b][pl.ds(k * L, L)] = plsc.load_gather(cid_v, [dv])

            base_c = ebase + c * E_CHUNK
            pltpu.async_copy(r3_v[b], out_hbm.at[pl.ds(base_c, E_CHUNK)],
                             osem[b])
            pltpu.async_copy(cs_v[b], osrc_hbm.at[pl.ds(base_c, E_CHUNK)],
                             osem[b])
            pltpu.async_copy(cd_v[b], odst_hbm.at[pl.ds(base_c, E_CHUNK)],
                             osem[b])

        return 0

    lax.fori_loop(0, N_GROUPS, group_body, 0)
    for b in range(NBUF):
        wait_writes(N_CHUNKS - NBUF + b, b)


@jax.jit
def _run(concept_ids, edge_index, edge_attr, concept_embedding,
         relation_embedding, W_msg, b_msg, W_self, W_upd, b_upd, W_lin,
         b_lin):
    src = edge_index[0]
    dst = edge_index[1]
    rel = edge_attr[:, 0].astype(jnp.int32)
    w = edge_attr[:, 1]

    idx_pad = jnp.concatenate(
        [concept_ids, jnp.zeros((GB - N_NODES,), jnp.int32)])
    x = _sc_gather_rows(concept_embedding, idx_pad)[:N_NODES]

    xm, rw2, rl2 = _tc_pre(x, W_msg[:D], W_msg[D:], W_lin[D:2 * D],
                           relation_embedding, b_msg.reshape(1, D))

    zeros = jnp.zeros((ROWS_PER_TILE, D), jnp.float32)
    parts = _sc_message(src, dst, rel, w, xm, rw2.reshape(-1), zeros)

    xs1, xs3 = _tc_post(parts, x, W_upd, W_self, b_upd.reshape(1, D),
                        W_lin[:D], W_lin[2 * D:], b_lin.reshape(1, D))

    enc, cs, cd = _sc_encode(src, dst, rel, w, xs1, xs3, rl2.reshape(-1),
                             concept_ids)
    triple_ids = jnp.stack([cs, rel, cd], axis=1)
    return enc, triple_ids


def kernel(concept_ids, edge_index, edge_attr, concept_embedding,
           relation_embedding, W_msg, b_msg, W_self, W_upd, b_upd, W_lin,
           b_lin):
    return _run(concept_ids, edge_index, edge_attr, concept_embedding,
                relation_embedding, W_msg, b_msg, W_self, W_upd, b_upd,
                W_lin, b_lin)


# speedup vs baseline: 1.3361x; 1.0859x over previous
import functools

import jax
import jax.numpy as jnp
from jax import lax
from jax.experimental import pallas as pl
from jax.experimental.pallas import tpu as pltpu
from jax.experimental.pallas import tpu_sc as plsc

D = 128
N_NODES = 10000
N_EDGES = 320000
N_REL = 38

NC = 2
NS = 16
NW = NC * NS
L = 16

E_PER_W = N_EDGES // NW
E_CHUNK = 80
N_CHUNKS = E_PER_W // E_CHUNK
NBUF = 5
N_GROUPS = N_CHUNKS // NBUF

M_CHUNK = 40
M_N_CHUNKS = E_PER_W // M_CHUNK
M_GROUPS = M_N_CHUNKS // NBUF

AGG_ROWS = 10240
ROWS_PER_TILE = AGG_ROWS // NS

GB = 10240
GB_PER_W = GB // NW
G_CHUNK = 80

_mesh = plsc.VectorSubcoreMesh(core_axis_name="c", subcore_axis_name="s")


def _wid():
    return lax.axis_index("s") * NC + lax.axis_index("c")


@functools.partial(
    pl.kernel,
    out_type=jax.ShapeDtypeStruct((GB, D), jnp.float32),
    mesh=_mesh,
    scratch_types=[
        pltpu.VMEM((GB_PER_W,), jnp.int32),
        pltpu.VMEM((G_CHUNK, D), jnp.float32),
        pltpu.SemaphoreType.DMA,
    ],
)
def _sc_gather_rows(table_hbm, idx_hbm, out_hbm, idx_v, rows_v, sem):
    wid = _wid()
    base = wid * GB_PER_W
    pltpu.sync_copy(idx_hbm.at[pl.ds(base, GB_PER_W)], idx_v)

    def body(c, _):
        pltpu.async_copy(
            table_hbm.at[idx_v.at[pl.ds(c * G_CHUNK, G_CHUNK)]], rows_v, sem
        ).wait()
        pltpu.sync_copy(rows_v, out_hbm.at[pl.ds(base + c * G_CHUNK, G_CHUNK)])
        return 0

    lax.fori_loop(0, GB_PER_W // G_CHUNK, body, 0)


def _tc_pre_body(x_ref, wm1_ref, wm2_ref, l2_ref, rel_ref, bm_ref, xm_ref,
                 rw2_ref, rl2_ref):
    xm_ref[...] = jnp.dot(x_ref[...], wm1_ref[...],
                          preferred_element_type=jnp.float32) + bm_ref[...]
    rw2_ref[...] = jnp.dot(rel_ref[...], wm2_ref[...],
                           preferred_element_type=jnp.float32)
    rl2_ref[...] = jnp.dot(rel_ref[...], l2_ref[...],
                           preferred_element_type=jnp.float32)


def _tc_pre(x, wm1, wm2, l2, rel_emb, bm):
    return pl.pallas_call(
        _tc_pre_body,
        grid=(10,),
        in_specs=[
            pl.BlockSpec((1000, D), lambda i: (i, 0)),
            pl.BlockSpec((D, D), lambda i: (0, 0)),
            pl.BlockSpec((D, D), lambda i: (0, 0)),
            pl.BlockSpec((D, D), lambda i: (0, 0)),
            pl.BlockSpec((N_REL, D), lambda i: (0, 0)),
            pl.BlockSpec((1, D), lambda i: (0, 0)),
        ],
        out_specs=[
            pl.BlockSpec((1000, D), lambda i: (i, 0)),
            pl.BlockSpec((N_REL, D), lambda i: (0, 0)),
            pl.BlockSpec((N_REL, D), lambda i: (0, 0)),
        ],
        out_shape=[
            jax.ShapeDtypeStruct((N_NODES, D), jnp.float32),
            jax.ShapeDtypeStruct((N_REL, D), jnp.float32),
            jax.ShapeDtypeStruct((N_REL, D), jnp.float32),
        ],
    )(x, wm1, wm2, l2, rel_emb, bm)


@functools.partial(
    pl.kernel,
    out_type=jax.ShapeDtypeStruct((NC, AGG_ROWS, D), jnp.float32),
    mesh=_mesh,
    scratch_types=[
        pltpu.VMEM((E_PER_W,), jnp.int32),
        [pltpu.VMEM((M_CHUNK,), jnp.int32) for _ in range(NBUF)],
        [pltpu.VMEM((M_CHUNK,), jnp.int32) for _ in range(NBUF)],
        [pltpu.VMEM((M_CHUNK,), jnp.float32) for _ in range(NBUF)],
        [pltpu.VMEM((M_CHUNK, D), jnp.float32) for _ in range(NBUF)],
        pltpu.VMEM((M_CHUNK, D), jnp.float32),
        pltpu.VMEM((N_REL * D,), jnp.float32),
        pltpu.VMEM_SHARED((AGG_ROWS, D), jnp.float32),
        [pltpu.SemaphoreType.DMA for _ in range(NBUF)],
        [pltpu.SemaphoreType.DMA for _ in range(NBUF)],
    ],
    compiler_params=pltpu.CompilerParams(needs_layout_passes=False),
)
def _sc_message(src_hbm, dst_hbm, rel_hbm, w_hbm, xm_hbm, rw2_hbm,
                zeros_hbm, out_hbm, src_v, dst_v, rel_v, w_v, rows_v, msg_v,
                rw2_v, agg_sh, gsem, ssem):
    core = lax.axis_index("c")
    sid = lax.axis_index("s")
    wid = sid * NC + core
    ebase = wid * E_PER_W

    pltpu.sync_copy(rw2_hbm, rw2_v)
    pltpu.sync_copy(src_hbm.at[pl.ds(ebase, E_PER_W)], src_v)
    pltpu.sync_copy(zeros_hbm, agg_sh.at[pl.ds(sid * ROWS_PER_TILE,
                                               ROWS_PER_TILE)])

    def issue(c, b):
        base = ebase + c * M_CHUNK
        pltpu.async_copy(xm_hbm.at[src_v.at[pl.ds(c * M_CHUNK, M_CHUNK)]],
                         rows_v[b], gsem[b])
        pltpu.async_copy(dst_hbm.at[pl.ds(base, M_CHUNK)], dst_v[b], ssem[b])
        pltpu.async_copy(rel_hbm.at[pl.ds(base, M_CHUNK)], rel_v[b], ssem[b])
        pltpu.async_copy(w_hbm.at[pl.ds(base, M_CHUNK)], w_v[b], ssem[b])

    for b in range(NBUF):
        issue(b, b)
    plsc.subcore_barrier()

    iota = lax.broadcasted_iota(jnp.int32, (L,), 0)

    def group_body(g, _):
        for b in range(NBUF):
            c = g * NBUF + b
            pltpu.make_async_copy(
                xm_hbm.at[src_v.at[pl.ds(c * M_CHUNK, M_CHUNK)]], rows_v[b],
                gsem[b]).wait()
            pltpu.make_async_copy(dst_hbm.at[pl.ds(0, M_CHUNK)], dst_v[b],
                                  ssem[b]).wait()
            pltpu.make_async_copy(rel_hbm.at[pl.ds(0, M_CHUNK)], rel_v[b],
                                  ssem[b]).wait()
            pltpu.make_async_copy(w_hbm.at[pl.ds(0, M_CHUNK)], w_v[b],
                                  ssem[b]).wait()

            @plsc.parallel_loop(0, M_CHUNK, unroll=4)
            def edge_body(e):
                ev = jnp.broadcast_to(e, (L,)).astype(jnp.int32)
                wv = plsc.load_gather(w_v[b], [ev])
                rv = plsc.load_gather(rel_v[b], [ev])
                rbase = rv * D + iota
                for j in range(D // L):
                    xmv = rows_v[b][e, pl.ds(j * L, L)]
                    relv = plsc.load_gather(rw2_v, [rbase + (j * L)])
                    msg_v[e, pl.ds(j * L, L)] = jnp.maximum(
                        xmv + wv * relv, 0.0)

            pltpu.sync_copy(msg_v, agg_sh.at[dst_v[b]], add=True)

            c5 = c + NBUF

            @pl.when(c5 < M_N_CHUNKS)
            def _():
                issue(c5, b)

        return 0

    lax.fori_loop(0, M_GROUPS, group_body, 0)
    plsc.subcore_barrier()
    pltpu.sync_copy(
        agg_sh.at[pl.ds(sid * ROWS_PER_TILE, ROWS_PER_TILE)],
        out_hbm.at[core].at[pl.ds(sid * ROWS_PER_TILE, ROWS_PER_TILE)],
    )


def _tc_post_body(parts_ref, x_ref, wu_ref, ws_ref, bu_ref, l1_ref, l3_ref,
                  bl_ref, xs1_ref, xs3_ref):
    agg = parts_ref[0] + parts_ref[1]
    x_out = jax.nn.relu(
        jnp.dot(agg, wu_ref[...], preferred_element_type=jnp.float32)
        + jnp.dot(x_ref[...], ws_ref[...], preferred_element_type=jnp.float32)
        + bu_ref[...]
    )
    xs1_ref[...] = jnp.dot(x_out, l1_ref[...],
                           preferred_element_type=jnp.float32) + bl_ref[...]
    xs3_ref[...] = jnp.dot(x_out, l3_ref[...],
                           preferred_element_type=jnp.float32)


def _tc_post(parts, x, wu, ws, bu, l1, l3, bl):
    return pl.pallas_call(
        _tc_post_body,
        grid=(10,),
        in_specs=[
            pl.BlockSpec((NC, 1000, D), lambda i: (0, i, 0)),
            pl.BlockSpec((1000, D), lambda i: (i, 0)),
            pl.BlockSpec((D, D), lambda i: (0, 0)),
            pl.BlockSpec((D, D), lambda i: (0, 0)),
            pl.BlockSpec((1, D), lambda i: (0, 0)),
            pl.BlockSpec((D, D), lambda i: (0, 0)),
            pl.BlockSpec((D, D), lambda i: (0, 0)),
            pl.BlockSpec((1, D), lambda i: (0, 0)),
        ],
        out_specs=[
            pl.BlockSpec((1000, D), lambda i: (i, 0)),
            pl.BlockSpec((1000, D), lambda i: (i, 0)),
        ],
        out_shape=[
            jax.ShapeDtypeStruct((N_NODES, D), jnp.float32),
            jax.ShapeDtypeStruct((N_NODES, D), jnp.float32),
        ],
    )(parts, x, wu, ws, bu, l1, l3, bl)


@functools.partial(
    pl.kernel,
    out_type=[
        jax.ShapeDtypeStruct((N_EDGES, D), jnp.float32),
        jax.ShapeDtypeStruct((N_EDGES,), jnp.int32),
        jax.ShapeDtypeStruct((N_EDGES,), jnp.int32),
    ],
    mesh=_mesh,
    scratch_types=[
        pltpu.VMEM((E_PER_W,), jnp.int32),
        [pltpu.VMEM((E_CHUNK,), jnp.int32) for _ in range(NBUF)],
        [pltpu.VMEM((E_CHUNK,), jnp.int32) for _ in range(NBUF)],
        [pltpu.VMEM((E_CHUNK,), jnp.float32) for _ in range(NBUF)],
        [pltpu.VMEM((E_CHUNK, D), jnp.float32) for _ in range(NBUF)],
        [pltpu.VMEM((E_CHUNK, D), jnp.float32) for _ in range(NBUF)],
        pltpu.VMEM((N_REL * D,), jnp.float32),
        pltpu.VMEM((N_NODES,), jnp.int32),
        [pltpu.VMEM((E_CHUNK,), jnp.int32) for _ in range(NBUF)],
        [pltpu.VMEM((E_CHUNK,), jnp.int32) for _ in range(NBUF)],
        [pltpu.SemaphoreType.DMA for _ in range(NBUF)],
        [pltpu.SemaphoreType.DMA for _ in range(NBUF)],
        [pltpu.SemaphoreType.DMA for _ in range(NBUF)],
        [pltpu.SemaphoreType.DMA for _ in range(NBUF)],
        [pltpu.SemaphoreType.DMA for _ in range(NBUF)],
    ],
    compiler_params=pltpu.CompilerParams(needs_layout_passes=False),
)
def _sc_encode(src_hbm, dst_hbm, rel_hbm, w_hbm, xs1_hbm, xs3_hbm, rl2_hbm,
               cid_hbm, out_hbm, osrc_hbm, odst_hbm, src_v, dst_v, rel_v,
               w_v, r1_v, r3_v, rl2_v, cid_v, cs_v, cd_v, g1sem, g3sem,
               ssem, osem, dsem):
    wid = _wid()
    ebase = wid * E_PER_W

    pltpu.sync_copy(rl2_hbm, rl2_v)
    pltpu.sync_copy(cid_hbm, cid_v)
    pltpu.sync_copy(src_hbm.at[pl.ds(ebase, E_PER_W)], src_v)

    def prefetch_dst(c, b):
        pltpu.async_copy(dst_hbm.at[pl.ds(ebase + c * E_CHUNK, E_CHUNK)],
                         dst_v[b], dsem[b])

    def wait_dst(b):
        pltpu.make_async_copy(dst_hbm.at[pl.ds(0, E_CHUNK)], dst_v[b],
                              dsem[b]).wait()

    def issue(c, b):
        base = ebase + c * E_CHUNK
        pltpu.async_copy(xs1_hbm.at[src_v.at[pl.ds(c * E_CHUNK, E_CHUNK)]],
                         r1_v[b], g1sem[b])
        pltpu.async_copy(xs3_hbm.at[dst_v[b]], r3_v[b], g3sem[b])
        pltpu.async_copy(rel_hbm.at[pl.ds(base, E_CHUNK)], rel_v[b], ssem[b])
        pltpu.async_copy(w_hbm.at[pl.ds(base, E_CHUNK)], w_v[b], ssem[b])

    def wait_writes(c, b):
        base = ebase + c * E_CHUNK
        pltpu.make_async_copy(r3_v[b], out_hbm.at[pl.ds(base, E_CHUNK)],
                              osem[b]).wait()
        pltpu.make_async_copy(cs_v[b], osrc_hbm.at[pl.ds(base, E_CHUNK)],
                              osem[b]).wait()
        pltpu.make_async_copy(cd_v[b], odst_hbm.at[pl.ds(base, E_CHUNK)],
                              osem[b]).wait()

    for b in range(NBUF - 1):
        pltpu.sync_copy(dst_hbm.at[pl.ds(ebase + b * E_CHUNK, E_CHUNK)],
                        dst_v[b])
        issue(b, b)
    prefetch_dst(NBUF - 1, NBUF - 1)

    iota = lax.broadcasted_iota(jnp.int32, (L,), 0)

    def group_body(g, _):
        for b in range(NBUF):
            c = g * NBUF + b
            ca = c + NBUF - 1
            ba = (b + NBUF - 1) % NBUF

            @pl.when(ca < N_CHUNKS)
            def _():
                @pl.when(c >= 1)
                def _():
                    wait_writes(c - 1, ba)

                wait_dst(ba)
                issue(ca, ba)

            pltpu.make_async_copy(
                xs1_hbm.at[src_v.at[pl.ds(c * E_CHUNK, E_CHUNK)]], r1_v[b],
                g1sem[b]).wait()
            pltpu.make_async_copy(xs3_hbm.at[dst_v[b]], r3_v[b],
                                  g3sem[b]).wait()
            pltpu.make_async_copy(rel_hbm.at[pl.ds(0, E_CHUNK)], rel_v[b],
                                  ssem[b]).wait()
            pltpu.make_async_copy(w_hbm.at[pl.ds(0, E_CHUNK)], w_v[b],
                                  ssem[b]).wait()

            @plsc.parallel_loop(0, E_CHUNK, unroll=4)
            def edge_body(e):
                ev = jnp.broadcast_to(e, (L,)).astype(jnp.int32)
                wv = plsc.load_gather(w_v[b], [ev])
                rv = plsc.load_gather(rel_v[b], [ev])
                rbase = rv * D + iota
                for j in range(D // L):
                    a = r1_v[b][e, pl.ds(j * L, L)]
                    bb = r3_v[b][e, pl.ds(j * L, L)]
                    relv = plsc.load_gather(rl2_v, [rbase + (j * L)])
                    r3_v[b][e, pl.ds(j * L, L)] = a + wv * relv + bb

            for k in range(E_CHUNK // L):
                sv = src_v[pl.ds(c * E_CHUNK + k * L, L)]
                dv = dst_v[b][pl.ds(k * L, L)]
                cs_v[b][pl.ds(k * L, L)] = plsc.load_gather(cid_v, [sv])
                cd_v[b][pl.ds(k * L, L)] = plsc.load_gather(cid_v, [dv])

            base_c = ebase + c * E_CHUNK
            pltpu.async_copy(r3_v[b], out_hbm.at[pl.ds(base_c, E_CHUNK)],
                             osem[b])
            pltpu.async_copy(cs_v[b], osrc_hbm.at[pl.ds(base_c, E_CHUNK)],
                             osem[b])
            pltpu.async_copy(cd_v[b], odst_hbm.at[pl.ds(base_c, E_CHUNK)],
                             osem[b])

            @pl.when(c + NBUF < N_CHUNKS)
            def _():
                prefetch_dst(c + NBUF, b)

        return 0

    lax.fori_loop(0, N_GROUPS, group_body, 0)
    for b in range(NBUF):
        wait_writes(N_CHUNKS - NBUF + b, b)


@jax.jit
def _run(concept_ids, edge_index, edge_attr, concept_embedding,
         relation_embedding, W_msg, b_msg, W_self, W_upd, b_upd, W_lin,
         b_lin):
    src = edge_index[0]
    dst = edge_index[1]
    rel = edge_attr[:, 0].astype(jnp.int32)
    w = edge_attr[:, 1]

    idx_pad = jnp.concatenate(
        [concept_ids, jnp.zeros((GB - N_NODES,), jnp.int32)])
    x = _sc_gather_rows(concept_embedding, idx_pad)[:N_NODES]

    xm, rw2, rl2 = _tc_pre(x, W_msg[:D], W_msg[D:], W_lin[D:2 * D],
                           relation_embedding, b_msg.reshape(1, D))

    zeros = jnp.zeros((ROWS_PER_TILE, D), jnp.float32)
    parts = _sc_message(src, dst, rel, w, xm, rw2.reshape(-1), zeros)

    xs1, xs3 = _tc_post(parts, x, W_upd, W_self, b_upd.reshape(1, D),
                        W_lin[:D], W_lin[2 * D:], b_lin.reshape(1, D))

    enc, cs, cd = _sc_encode(src, dst, rel, w, xs1, xs3, rl2.reshape(-1),
                             concept_ids)
    triple_ids = jnp.stack([cs, rel, cd], axis=1)
    return enc, triple_ids


def kernel(concept_ids, edge_index, edge_attr, concept_embedding,
           relation_embedding, W_msg, b_msg, W_self, W_upd, b_upd, W_lin,
           b_lin):
    return _run(concept_ids, edge_index, edge_attr, concept_embedding,
                relation_embedding, W_msg, b_msg, W_self, W_upd, b_upd,
                W_lin, b_lin)
